# baseline (speedup 1.0000x reference)
import functools
import math

import jax
import jax.numpy as jnp
import numpy as np
from jax import lax
from jax.experimental import pallas as pl
from jax.experimental.pallas import tpu as pltpu
from jax.experimental.pallas import tpu_sc as plsc

N, E, D, H, WD, NW = 10000, 160000, 128, 8, 300, 512
DH = D // H

NC, NS = 2, 16
NWRK = NC * NS
CH = 128
NCHUNK = E // CH
MAXC = -(-NCHUNK // NWRK)

_INTERPRET = False
_USE_SC_GATHER = True
_USE_SC_SCATTER = True
_SCATTER_PHASE = 4
_USE_BARRIER = True


def _sc_mesh():
    return plsc.VectorSubcoreMesh(core_axis_name="c", subcore_axis_name="s",
                                  num_cores=NC, num_subcores=NS)


def _gather_mul(kh, qh, src, dst):
    @functools.partial(
        pl.kernel,
        out_type=jax.ShapeDtypeStruct((E, D), jnp.float32),
        mesh=_sc_mesh(),
        scratch_types=[
            pltpu.VMEM((CH,), jnp.int32),
            pltpu.VMEM((CH,), jnp.int32),
            pltpu.VMEM((CH,), jnp.int32),
            pltpu.VMEM((CH,), jnp.int32),
            pltpu.VMEM((CH, D), jnp.float32),
            pltpu.VMEM((CH, D), jnp.float32),
            pltpu.VMEM((CH, D), jnp.float32),
            pltpu.VMEM((CH, D), jnp.float32),
            pltpu.VMEM((CH, D), jnp.float32),
            pltpu.SemaphoreType.DMA,
            pltpu.SemaphoreType.DMA,
            pltpu.SemaphoreType.DMA,
            pltpu.SemaphoreType.DMA,
        ],
    )
    def k(kh_h, qh_h, src_h, dst_h, g_h,
          si0, di0, si1, di1, kb0, qb0, kb1, qb1, gb,
          s10, s20, s11, s21):
        wid = lax.axis_index("s") * NC + lax.axis_index("c")
        bufs = ((si0, di0, kb0, qb0, s10, s20),
                (si1, di1, kb1, qb1, s11, s21))

        def prefetch(c, b):
            si, di, kb, qb, s1, s2 = b

            @pl.when(c < NCHUNK)
            def _():
                base = pl.multiple_of(c * CH, CH)
                pltpu.sync_copy(src_h.at[pl.ds(base, CH)], si)
                pltpu.sync_copy(dst_h.at[pl.ds(base, CH)], di)
                pltpu.async_copy(kh_h.at[si], kb, s1)
                pltpu.async_copy(qh_h.at[di], qb, s2)

        def process(c, b):
            si, di, kb, qb, s1, s2 = b

            @pl.when(c < NCHUNK)
            def _():
                base = pl.multiple_of(c * CH, CH)
                pltpu.make_async_copy(kh_h.at[si], kb, s1).wait()
                pltpu.make_async_copy(qh_h.at[di], qb, s2).wait()

                def e_body(i, _):
                    for h in range(H):
                        s = pl.ds(h * DH, DH)
                        gb[i, s] = kb[i, s] * qb[i, s]
                    return 0

                lax.fori_loop(0, CH, e_body, 0)
                pltpu.sync_copy(gb, g_h.at[pl.ds(base, CH)])

        prefetch(wid, bufs[0])

        def loop_body(j2, _):
            c0 = wid + (2 * j2) * NWRK
            prefetch(c0 + NWRK, bufs[1])
            process(c0, bufs[0])
            prefetch(c0 + 2 * NWRK, bufs[0])
            process(c0 + NWRK, bufs[1])
            return 0

        lax.fori_loop(0, MAXC // 2, loop_body, 0)

    return k(kh, qh, src, dst)


CH2 = 64
NCHUNK2 = E // CH2
MAXC2 = -(-NCHUNK2 // NWRK)


def _scatter_seg(vhx, src, dst, sc16):
    @functools.partial(
        pl.kernel,
        out_type=jax.ShapeDtypeStruct((NC, N, D), jnp.float32),
        mesh=_sc_mesh(),
        scratch_types=[
            pltpu.VMEM((CH2,), jnp.int32),
            pltpu.VMEM((CH2,), jnp.int32),
            pltpu.VMEM((CH2,), jnp.int32),
            pltpu.VMEM((CH2,), jnp.int32),
            pltpu.VMEM((CH2, D), jnp.float32),
            pltpu.VMEM((CH2, D), jnp.float32),
            pltpu.VMEM((CH2, DH), jnp.float32),
            pltpu.VMEM((CH2, DH), jnp.float32),
            pltpu.VMEM((CH2,), jnp.int32),
            pltpu.VMEM_SHARED((N, D), jnp.float32),
            pltpu.SemaphoreType.DMA,
            pltpu.SemaphoreType.DMA,
        ],
    )
    def k(vh_h, src_h, dst_h, sc_h, wv_o,
          si0, di0, si1, di1, vb0, vb1, scb0, scb1, ri, wv_acc, sem0, sem1):
        cid = lax.axis_index("c")
        sid = lax.axis_index("s")
        wid = sid * NC + cid
        vb = vb0

        def z_body(i, _):
            zv = jnp.zeros((DH,), jnp.float32)
            for h in range(D // DH):
                vb[i, pl.ds(h * DH, DH)] = zv
            return 0

        lax.fori_loop(0, CH2, z_body, 0)

        NZCH2 = -(-N // CH2)
        NZCH = N // CH2
        TAIL = N - NZCH * CH2
        NJ = -(-NZCH2 // NS)

        def fill_ri(c):
            for kk in range(CH2 // DH):
                ri[pl.ds(kk * DH, DH)] = jnp.minimum(
                    lax.iota(jnp.int32, DH) + (c * CH2 + kk * DH), N - 1)

        for j in range(NJ):
            c = sid + j * NS

            @pl.when(c < NZCH2)
            def _():
                fill_ri(c)
                pltpu.sync_copy(vb, wv_acc.at[ri])

        plsc.subcore_barrier()

        dbufs = ((si0, di0, vb0, scb0, sem0), (si1, di1, vb1, scb1, sem1))

        def prefetch(c, b):
            si, di, vbx, scbx, sem = b

            @pl.when(c < NCHUNK2)
            def _():
                base = pl.multiple_of(c * CH2, CH2)
                pltpu.sync_copy(src_h.at[pl.ds(base, CH2)], si)
                pltpu.sync_copy(dst_h.at[pl.ds(base, CH2)], di)
                pltpu.async_copy(vh_h.at[si], vbx, sem)
                pltpu.sync_copy(sc_h.at[pl.ds(base, CH2)], scbx)

        def process(c, b):
            si, di, vbx, scbx, sem = b

            @pl.when(c < NCHUNK2)
            def _():
                pltpu.make_async_copy(vh_h.at[si], vbx, sem).wait()

                def e_body(i, _):
                    row = scbx[i, :]
                    dnums = lax.GatherDimensionNumbers(
                        offset_dims=(), collapsed_slice_dims=(0,),
                        start_index_map=(0,))
                    for h in range(H):
                        hh = jnp.full((DH, 1), h, dtype=jnp.int32)
                        bc = lax.gather(
                            row, hh, dnums, (1,),
                            mode=lax.GatherScatterMode.PROMISE_IN_BOUNDS)
                        s = pl.ds(h * DH, DH)
                        vbx[i, s] = vbx[i, s] * bc
                    return 0

                lax.fori_loop(0, CH2, e_body, 0)
                pltpu.sync_copy(vbx, wv_acc.at[di], add=True)

        prefetch(wid, dbufs[0])

        def loop_body(j2, _):
            c0 = wid + (2 * j2) * NWRK
            prefetch(c0 + NWRK, dbufs[1])
            process(c0, dbufs[0])
            prefetch(c0 + 2 * NWRK, dbufs[0])
            process(c0 + NWRK, dbufs[1])
            return 0

        lax.fori_loop(0, (MAXC2 + 1) // 2, loop_body, 0)
        plsc.subcore_barrier()

        for j in range(NJ):
            c = sid + j * NS

            @pl.when(c < NZCH)
            def _():
                fill_ri(c)
                pltpu.sync_copy(wv_acc.at[ri], vb)
                r0 = pl.multiple_of(c * CH2, CH2)
                pltpu.sync_copy(vb, wv_o.at[cid, pl.ds(r0, CH2)])

            @pl.when(c == NZCH)
            def _():
                fill_ri(c)
                pltpu.sync_copy(wv_acc.at[ri], vb)
                pltpu.sync_copy(vb.at[pl.ds(0, TAIL)],
                                wv_o.at[cid, pl.ds(NZCH * CH2, TAIL)])

    return k(vhx, src, dst, sc16)


def _scatter_z(dst, sc16):
    @functools.partial(
        pl.kernel,
        out_type=jax.ShapeDtypeStruct((NC, N, D), jnp.float32),
        mesh=_sc_mesh(),
        scratch_types=[
            pltpu.VMEM((CH,), jnp.int32),
            pltpu.VMEM((CH, D), jnp.float32),
            pltpu.VMEM((CH, DH), jnp.float32),
            pltpu.VMEM((CH,), jnp.int32),
            pltpu.VMEM_SHARED((N, D), jnp.float32),
        ],
    )
    def k(dst_h, sc_h, z_o, di, zb, scb, ri, z_acc):
        cid = lax.axis_index("c")
        sid = lax.axis_index("s")
        wid = sid * NC + cid

        def z_body(i, _):
            zv = jnp.zeros((DH,), jnp.float32)
            for h in range(D // DH):
                zb[i, pl.ds(h * DH, DH)] = zv
            return 0

        lax.fori_loop(0, CH, z_body, 0)

        NZCH2 = -(-N // CH)
        NZCH = N // CH
        TAIL = N - NZCH * CH

        def fill_ri(c):
            for kk in range(CH // DH):
                ri[pl.ds(kk * DH, DH)] = jnp.minimum(
                    lax.iota(jnp.int32, DH) + (c * CH + kk * DH), N - 1)

        for j in range(5):
            c = sid + j * NS

            @pl.when(c < NZCH2)
            def _():
                fill_ri(c)
                pltpu.sync_copy(zb, z_acc.at[ri])

        plsc.subcore_barrier()

        def chunk_body(j, _):
            c = wid + j * NWRK

            @pl.when(c < NCHUNK)
            def _():
                base = pl.multiple_of(c * CH, CH)
                pltpu.sync_copy(dst_h.at[pl.ds(base, CH)], di)
                pltpu.sync_copy(sc_h.at[pl.ds(base, CH)], scb)

                def e_body(i, _):
                    zb[i, pl.ds(0, DH)] = scb[i, :]
                    return 0

                lax.fori_loop(0, CH, e_body, 0)
                pltpu.sync_copy(zb, z_acc.at[di], add=True)
            return 0

        lax.fori_loop(0, MAXC, chunk_body, 0)
        plsc.subcore_barrier()

        for j in range(5):
            c = sid + j * NS

            @pl.when(c < NZCH)
            def _():
                fill_ri(c)
                pltpu.sync_copy(z_acc.at[ri], zb)
                r0 = pl.multiple_of(c * CH, CH)
                pltpu.sync_copy(zb, z_o.at[cid, pl.ds(r0, CH)])

            @pl.when(c == NZCH)
            def _():
                fill_ri(c)
                pltpu.sync_copy(z_acc.at[ri], zb)
                pltpu.sync_copy(zb.at[pl.ds(0, TAIL)],
                                z_o.at[cid, pl.ds(NZCH * CH, TAIL)])

    return k(dst, sc16)


def _ln(x, g, b, eps=1e-5):
    mu = jnp.mean(x, axis=-1, keepdims=True)
    var = jnp.mean((x - mu) ** 2, axis=-1, keepdims=True)
    return (x - mu) / jnp.sqrt(var + eps) * g + b


def _qkv_body(nf, wq, wk, wv, q_o, k_o, v_o):
    x = nf[...]
    q_o[...] = jnp.dot(x, wq[...], preferred_element_type=jnp.float32)
    k_o[...] = jnp.dot(x, wk[...], preferred_element_type=jnp.float32)
    v_o[...] = jnp.dot(x, wv[...], preferred_element_type=jnp.float32)


def _qkv(nf, wq, wk, wv, bn=2000):
    grid = N // bn
    wspec = pl.BlockSpec((D, D), lambda i: (0, 0))
    rspec = pl.BlockSpec((bn, D), lambda i: (i, 0))
    return pl.pallas_call(
        _qkv_body,
        grid=(grid,),
        in_specs=[rspec, wspec, wspec, wspec],
        out_specs=[rspec, rspec, rspec],
        out_shape=[jax.ShapeDtypeStruct((N, D), jnp.float32)] * 3,
        interpret=_INTERPRET,
    )(nf, wq, wk, wv)


def _edge_body(ef_r, g_r, wpe_r, oew_r, oeb_r, l1g_r, l1b_r,
               f1w_r, f1b_r, f2w_r, f2b_r, l2g_r, l2b_r, c16_r,
               e_o, sc_o):
    ef = ef_r[...]
    pe = jnp.dot(ef, wpe_r[...], preferred_element_type=jnp.float32)
    score = g_r[...] * pe
    s16 = jnp.dot(score, c16_r[...], preferred_element_type=jnp.float32)
    hmask = (jax.lax.broadcasted_iota(jnp.int32, s16.shape, 1) < H).astype(
        jnp.float32)
    sc_o[...] = jnp.exp(jnp.clip(s16, -5.0, 5.0)) * hmask
    e = jnp.dot(score, oew_r[...], preferred_element_type=jnp.float32) + oeb_r[...]
    e = _ln(ef + e, l1g_r[...], l1b_r[...])
    e2 = jnp.maximum(
        jnp.dot(e, f1w_r[...], preferred_element_type=jnp.float32) + f1b_r[...], 0.0)
    e2 = jnp.dot(e2, f2w_r[...], preferred_element_type=jnp.float32) + f2b_r[...]
    e_o[...] = _ln(e + e2, l2g_r[...], l2b_r[...])


def _edge_dense(ef, g, p, c16, be=2000):
    grid = E // be
    rspec = pl.BlockSpec((be, D), lambda i: (i, 0))
    full = lambda *s: pl.BlockSpec(s, lambda i: (0,) * len(s))
    return pl.pallas_call(
        _edge_body,
        grid=(grid,),
        in_specs=[rspec, rspec, full(D, D), full(D, D), full(D,),
                  full(D,), full(D,), full(D, 2 * D), full(2 * D,),
                  full(2 * D, D), full(D,), full(D,), full(D,), full(D, DH)],
        out_specs=[rspec, pl.BlockSpec((be, DH), lambda i: (i, 0))],
        out_shape=[jax.ShapeDtypeStruct((E, D), jnp.float32),
                   jax.ShapeDtypeStruct((E, DH), jnp.float32)],
        interpret=_INTERPRET,
    )(ef, g, p['Wpe'], p['OeW'], p['Oeb'], p['ln1e_g'], p['ln1e_b'],
      p['F1eW'], p['F1eb'], p['F2eW'], p['F2eb'], p['ln2e_g'], p['ln2e_b'], c16)


def _node_body(nf_r, wv_r, zx_r, e16_r, wfp_r,
               ohw_r, ohb_r, l1g_r, l1b_r, f1w_r, f1b_r, f2w_r, f2b_r,
               l2g_r, l2b_r,
               cawq_r, cawk_r, cawv_r, cawatt_r, cl1g_r, cl1b_r,
               cf1_r, cf2_r, cl2g_r, cl2b_r, hn_o):
    nf = nf_r[...]
    wv = wv_r[0] + wv_r[1]
    z16 = (zx_r[0] + zx_r[1])[:, :DH]
    z = jnp.dot(z16, e16_r[...], preferred_element_type=jnp.float32)
    h_attn = wv / (z + 1e-6)
    h = jnp.dot(h_attn, ohw_r[...], preferred_element_type=jnp.float32) + ohb_r[...]
    h = _ln(nf + h, l1g_r[...], l1b_r[...])
    h2 = jnp.maximum(
        jnp.dot(h, f1w_r[...], preferred_element_type=jnp.float32) + f1b_r[...], 0.0)
    h2 = jnp.dot(h2, f2w_r[...], preferred_element_type=jnp.float32) + f2b_r[...]
    h = _ln(h + h2, l2g_r[...], l2b_r[...])
    wfp = wfp_r[...]
    k = jnp.dot(wfp, cawk_r[...], preferred_element_type=jnp.float32)
    v = jnp.dot(wfp, cawv_r[...], preferred_element_type=jnp.float32)
    q = jnp.dot(h, cawq_r[...], preferred_element_type=jnp.float32)
    logits = jnp.dot(q, k.T, preferred_element_type=jnp.float32) * (
        1.0 / math.sqrt(D))
    logits = logits - jnp.max(logits, axis=-1, keepdims=True)
    logits = jnp.exp(logits)
    logits = logits / jnp.sum(logits, axis=-1, keepdims=True)
    att = jnp.dot(jnp.dot(logits, v, preferred_element_type=jnp.float32),
                  cawatt_r[...], preferred_element_type=jnp.float32)
    hn = _ln(h + att, cl1g_r[...], cl1b_r[...])
    ff = jnp.maximum(jnp.dot(hn, cf1_r[...], preferred_element_type=jnp.float32), 0.0)
    ff = jnp.dot(ff, cf2_r[...], preferred_element_type=jnp.float32)
    hn_o[...] = _ln(hn + ff, cl2g_r[...], cl2b_r[...])


def _node_post(nf, wv, zx, e16, wfp, p, bn=2000):
    grid = N // bn
    rspec = pl.BlockSpec((bn, D), lambda i: (i, 0))
    full = lambda *s: pl.BlockSpec(s, lambda i: (0,) * len(s))
    WDP = wfp.shape[1]
    pspec = pl.BlockSpec((NC, bn, D), lambda i: (0, i, 0))
    return pl.pallas_call(
        _node_body,
        grid=(grid,),
        in_specs=[rspec, pspec, pspec,
                  full(DH, D), full(NW, WDP),
                  full(D, D), full(D,), full(D,), full(D,),
                  full(D, 2 * D), full(2 * D,), full(2 * D, D), full(D,),
                  full(D,), full(D,),
                  full(D, D), full(WDP, D), full(WDP, D), full(D, D),
                  full(D,), full(D,),
                  full(D, 2 * D), full(2 * D, D), full(D,), full(D,)],
        out_specs=rspec,
        out_shape=jax.ShapeDtypeStruct((N, D), jnp.float32),
        interpret=_INTERPRET,
    )(nf, wv, zx, e16, wfp,
      p['OhW'], p['Ohb'], p['ln1h_g'], p['ln1h_b'],
      p['F1hW'], p['F1hb'], p['F2hW'], p['F2hb'], p['ln2h_g'], p['ln2h_b'],
      p['caWq'], p['_caWk_p'], p['_caWv_p'], p['caWatt'],
      p['ca_ln1_g'], p['ca_ln1_b'], p['caF1'], p['caF2'],
      p['ca_ln2_g'], p['ca_ln2_b'])


def kernel(node_feats, edge_feats, word_feats, edge_index, params):
    p = dict(params)
    src = edge_index[0]
    dst = edge_index[1]
    WDP = 384
    wfp = jnp.pad(word_feats, ((0, 0), (0, WDP - WD)))
    p['_caWk_p'] = jnp.pad(p['caWk'], ((0, WDP - WD), (0, 0)))
    p['_caWv_p'] = jnp.pad(p['caWv'], ((0, WDP - WD), (0, 0)))
    wq_scaled = p['WQ'] * (1.0 / math.sqrt(DH))
    c16 = (jnp.arange(D)[:, None] // DH == jnp.arange(DH)[None, :]).astype(
        jnp.float32)
    e16 = c16.T

    qh, kh, vh = _qkv(node_feats, wq_scaled, p['WK'], p['WV'])

    if _USE_SC_GATHER:
        g = _gather_mul(kh, qh, src, dst)
    else:
        g = kh[src] * qh[dst]

    e_out, sc16 = _edge_dense(edge_feats, g, p, c16)

    wv_p = _scatter_seg(vh, src, dst, sc16)
    z_p = _scatter_z(dst, sc16)

    hn = _node_post(node_feats, wv_p, z_p, e16, wfp, p)
    return (hn, e_out)


# speedup vs baseline: 1.0697x; 1.0697x over previous
import functools
import math

import jax
import jax.numpy as jnp
from jax import lax
from jax.experimental import pallas as pl
from jax.experimental.pallas import tpu as pltpu
from jax.experimental.pallas import tpu_sc as plsc

N, E, D, H, WD, NW = 10000, 160000, 128, 8, 300, 512
DH = D // H

NC, NS = 2, 16
NWRK = NC * NS
CH = 128
NCHUNK = E // CH
MAXC = -(-NCHUNK // NWRK)

def _sc_mesh():
    return plsc.VectorSubcoreMesh(core_axis_name="c", subcore_axis_name="s",
                                  num_cores=NC, num_subcores=NS)


def _gather_mul(kh, qh, src, dst):
    @functools.partial(
        pl.kernel,
        out_type=jax.ShapeDtypeStruct((E, D), jnp.float32),
        mesh=_sc_mesh(),
        scratch_types=[
            pltpu.VMEM((CH,), jnp.int32),
            pltpu.VMEM((CH,), jnp.int32),
            pltpu.VMEM((CH,), jnp.int32),
            pltpu.VMEM((CH,), jnp.int32),
            pltpu.VMEM((CH, D), jnp.float32),
            pltpu.VMEM((CH, D), jnp.float32),
            pltpu.VMEM((CH, D), jnp.float32),
            pltpu.VMEM((CH, D), jnp.float32),
            pltpu.VMEM((CH, D), jnp.float32),
            pltpu.SemaphoreType.DMA,
            pltpu.SemaphoreType.DMA,
            pltpu.SemaphoreType.DMA,
            pltpu.SemaphoreType.DMA,
        ],
    )
    def k(kh_h, qh_h, src_h, dst_h, g_h,
          si0, di0, si1, di1, kb0, qb0, kb1, qb1, gb,
          s10, s20, s11, s21):
        wid = lax.axis_index("s") * NC + lax.axis_index("c")
        bufs = ((si0, di0, kb0, qb0, s10, s20),
                (si1, di1, kb1, qb1, s11, s21))

        def prefetch(c, b):
            si, di, kb, qb, s1, s2 = b

            @pl.when(c < NCHUNK)
            def _():
                base = pl.multiple_of(c * CH, CH)
                pltpu.sync_copy(src_h.at[pl.ds(base, CH)], si)
                pltpu.sync_copy(dst_h.at[pl.ds(base, CH)], di)
                pltpu.async_copy(kh_h.at[si], kb, s1)
                pltpu.async_copy(qh_h.at[di], qb, s2)

        def process(c, b):
            si, di, kb, qb, s1, s2 = b

            @pl.when(c < NCHUNK)
            def _():
                base = pl.multiple_of(c * CH, CH)
                pltpu.make_async_copy(kh_h.at[si], kb, s1).wait()
                pltpu.make_async_copy(qh_h.at[di], qb, s2).wait()

                def e_body(i, _):
                    for h in range(H):
                        s = pl.ds(h * DH, DH)
                        gb[i, s] = kb[i, s] * qb[i, s]
                    return 0

                lax.fori_loop(0, CH, e_body, 0)
                pltpu.sync_copy(gb, g_h.at[pl.ds(base, CH)])

        prefetch(wid, bufs[0])

        def loop_body(j2, _):
            c0 = wid + (2 * j2) * NWRK
            prefetch(c0 + NWRK, bufs[1])
            process(c0, bufs[0])
            prefetch(c0 + 2 * NWRK, bufs[0])
            process(c0 + NWRK, bufs[1])
            return 0

        lax.fori_loop(0, MAXC // 2, loop_body, 0)

    return k(kh, qh, src, dst)


def _scatter_seg(vhx, src, dst, sc16):
    @functools.partial(
        pl.kernel,
        out_type=jax.ShapeDtypeStruct((NC, N, D), jnp.float32),
        mesh=_sc_mesh(),
        scratch_types=[
            pltpu.VMEM((CH,), jnp.int32),
            pltpu.VMEM((CH,), jnp.int32),
            pltpu.VMEM((CH, D), jnp.float32),
            pltpu.VMEM((CH, DH), jnp.float32),
            pltpu.VMEM((CH,), jnp.int32),
            pltpu.VMEM_SHARED((N, D), jnp.float32),
            pltpu.SemaphoreType.DMA,
        ],
    )
    def k(vh_h, src_h, dst_h, sc_h, wv_o, si, di, vb, scb, ri, wv_acc, sem):
        cid = lax.axis_index("c")
        sid = lax.axis_index("s")
        wid = sid * NC + cid

        def z_body(i, _):
            zv = jnp.zeros((DH,), jnp.float32)
            for h in range(D // DH):
                vb[i, pl.ds(h * DH, DH)] = zv
            return 0

        lax.fori_loop(0, CH, z_body, 0)

        NZCH2 = -(-N // CH)
        NZCH = N // CH
        TAIL = N - NZCH * CH

        def fill_ri(c):
            for kk in range(CH // DH):
                ri[pl.ds(kk * DH, DH)] = jnp.minimum(
                    lax.iota(jnp.int32, DH) + (c * CH + kk * DH), N - 1)

        for j in range(5):
            c = sid + j * NS

            @pl.when(c < NZCH2)
            def _():
                fill_ri(c)
                pltpu.sync_copy(vb, wv_acc.at[ri])

        plsc.subcore_barrier()

        def chunk_body(j, _):
            c = wid + j * NWRK

            @pl.when(c < NCHUNK)
            def _():
                base = pl.multiple_of(c * CH, CH)
                pltpu.sync_copy(src_h.at[pl.ds(base, CH)], si)
                pltpu.sync_copy(dst_h.at[pl.ds(base, CH)], di)
                cp = pltpu.async_copy(vh_h.at[si], vb, sem)
                pltpu.sync_copy(sc_h.at[pl.ds(base, CH)], scb)
                cp.wait()

                def e_body(i, _):
                    row = scb[i, :]
                    dnums = lax.GatherDimensionNumbers(
                        offset_dims=(), collapsed_slice_dims=(0,),
                        start_index_map=(0,))
                    for h in range(H):
                        hh = jnp.full((DH, 1), h, dtype=jnp.int32)
                        bc = lax.gather(
                            row, hh, dnums, (1,),
                            mode=lax.GatherScatterMode.PROMISE_IN_BOUNDS)
                        s = pl.ds(h * DH, DH)
                        vb[i, s] = vb[i, s] * bc
                    return 0

                lax.fori_loop(0, CH, e_body, 0)
                pltpu.sync_copy(vb, wv_acc.at[di], add=True)
            return 0

        lax.fori_loop(0, MAXC, chunk_body, 0)
        plsc.subcore_barrier()

        for j in range(5):
            c = sid + j * NS

            @pl.when(c < NZCH)
            def _():
                fill_ri(c)
                pltpu.sync_copy(wv_acc.at[ri], vb)
                r0 = pl.multiple_of(c * CH, CH)
                pltpu.sync_copy(vb, wv_o.at[cid, pl.ds(r0, CH)])

            @pl.when(c == NZCH)
            def _():
                fill_ri(c)
                pltpu.sync_copy(wv_acc.at[ri], vb)
                pltpu.sync_copy(vb.at[pl.ds(0, TAIL)],
                                wv_o.at[cid, pl.ds(NZCH * CH, TAIL)])

    return k(vhx, src, dst, sc16)


def _scatter_z(dst, sc16):
    @functools.partial(
        pl.kernel,
        out_type=jax.ShapeDtypeStruct((NC, N, D), jnp.float32),
        mesh=_sc_mesh(),
        scratch_types=[
            pltpu.VMEM((CH,), jnp.int32),
            pltpu.VMEM((CH, D), jnp.float32),
            pltpu.VMEM((CH, DH), jnp.float32),
            pltpu.VMEM((CH,), jnp.int32),
            pltpu.VMEM_SHARED((N, D), jnp.float32),
        ],
    )
    def k(dst_h, sc_h, z_o, di, zb, scb, ri, z_acc):
        cid = lax.axis_index("c")
        sid = lax.axis_index("s")
        wid = sid * NC + cid

        def z_body(i, _):
            zv = jnp.zeros((DH,), jnp.float32)
            for h in range(D // DH):
                zb[i, pl.ds(h * DH, DH)] = zv
            return 0

        lax.fori_loop(0, CH, z_body, 0)

        NZCH2 = -(-N // CH)
        NZCH = N // CH
        TAIL = N - NZCH * CH

        def fill_ri(c):
            for kk in range(CH // DH):
                ri[pl.ds(kk * DH, DH)] = jnp.minimum(
                    lax.iota(jnp.int32, DH) + (c * CH + kk * DH), N - 1)

        for j in range(5):
            c = sid + j * NS

            @pl.when(c < NZCH2)
            def _():
                fill_ri(c)
                pltpu.sync_copy(zb, z_acc.at[ri])

        plsc.subcore_barrier()

        def chunk_body(j, _):
            c = wid + j * NWRK

            @pl.when(c < NCHUNK)
            def _():
                base = pl.multiple_of(c * CH, CH)
                pltpu.sync_copy(dst_h.at[pl.ds(base, CH)], di)
                pltpu.sync_copy(sc_h.at[pl.ds(base, CH)], scb)

                def e_body(i, _):
                    zb[i, pl.ds(0, DH)] = scb[i, :]
                    return 0

                lax.fori_loop(0, CH, e_body, 0)
                pltpu.sync_copy(zb, z_acc.at[di], add=True)
            return 0

        lax.fori_loop(0, MAXC, chunk_body, 0)
        plsc.subcore_barrier()

        for j in range(5):
            c = sid + j * NS

            @pl.when(c < NZCH)
            def _():
                fill_ri(c)
                pltpu.sync_copy(z_acc.at[ri], zb)
                r0 = pl.multiple_of(c * CH, CH)
                pltpu.sync_copy(zb, z_o.at[cid, pl.ds(r0, CH)])

            @pl.when(c == NZCH)
            def _():
                fill_ri(c)
                pltpu.sync_copy(z_acc.at[ri], zb)
                pltpu.sync_copy(zb.at[pl.ds(0, TAIL)],
                                z_o.at[cid, pl.ds(NZCH * CH, TAIL)])

    return k(dst, sc16)


def _ln(x, g, b, eps=1e-5):
    mu = jnp.mean(x, axis=-1, keepdims=True)
    var = jnp.mean((x - mu) ** 2, axis=-1, keepdims=True)
    return (x - mu) / jnp.sqrt(var + eps) * g + b


def _qkv_body(nf, wq, wk, wv, q_o, k_o, v_o):
    x = nf[...]
    q_o[...] = jnp.dot(x, wq[...], preferred_element_type=jnp.float32)
    k_o[...] = jnp.dot(x, wk[...], preferred_element_type=jnp.float32)
    v_o[...] = jnp.dot(x, wv[...], preferred_element_type=jnp.float32)


def _qkv(nf, wq, wk, wv, bn=2000):
    grid = N // bn
    wspec = pl.BlockSpec((D, D), lambda i: (0, 0))
    rspec = pl.BlockSpec((bn, D), lambda i: (i, 0))
    return pl.pallas_call(
        _qkv_body,
        grid=(grid,),
        in_specs=[rspec, wspec, wspec, wspec],
        out_specs=[rspec, rspec, rspec],
        out_shape=[jax.ShapeDtypeStruct((N, D), jnp.float32)] * 3,
    )(nf, wq, wk, wv)


def _edge_body(ef_r, g_r, wpe_r, oew_r, oeb_r, l1g_r, l1b_r,
               f1w_r, f1b_r, f2w_r, f2b_r, l2g_r, l2b_r, c16_r,
               e_o, sc_o):
    ef = ef_r[...]
    pe = jnp.dot(ef, wpe_r[...], preferred_element_type=jnp.float32)
    score = g_r[...] * pe
    s16 = jnp.dot(score, c16_r[...], preferred_element_type=jnp.float32)
    hmask = (jax.lax.broadcasted_iota(jnp.int32, s16.shape, 1) < H).astype(
        jnp.float32)
    sc_o[...] = jnp.exp(jnp.clip(s16, -5.0, 5.0)) * hmask
    e = jnp.dot(score, oew_r[...], preferred_element_type=jnp.float32) + oeb_r[...]
    e = _ln(ef + e, l1g_r[...], l1b_r[...])
    e2 = jnp.maximum(
        jnp.dot(e, f1w_r[...], preferred_element_type=jnp.float32) + f1b_r[...], 0.0)
    e2 = jnp.dot(e2, f2w_r[...], preferred_element_type=jnp.float32) + f2b_r[...]
    e_o[...] = _ln(e + e2, l2g_r[...], l2b_r[...])


def _edge_dense(ef, g, p, c16, be=2000):
    grid = E // be
    rspec = pl.BlockSpec((be, D), lambda i: (i, 0))
    full = lambda *s: pl.BlockSpec(s, lambda i: (0,) * len(s))
    return pl.pallas_call(
        _edge_body,
        grid=(grid,),
        in_specs=[rspec, rspec, full(D, D), full(D, D), full(D,),
                  full(D,), full(D,), full(D, 2 * D), full(2 * D,),
                  full(2 * D, D), full(D,), full(D,), full(D,), full(D, DH)],
        out_specs=[rspec, pl.BlockSpec((be, DH), lambda i: (i, 0))],
        out_shape=[jax.ShapeDtypeStruct((E, D), jnp.float32),
                   jax.ShapeDtypeStruct((E, DH), jnp.float32)],
    )(ef, g, p['Wpe'], p['OeW'], p['Oeb'], p['ln1e_g'], p['ln1e_b'],
      p['F1eW'], p['F1eb'], p['F2eW'], p['F2eb'], p['ln2e_g'], p['ln2e_b'], c16)


def _node_body(nf_r, wv_r, zx_r, e16_r, wfp_r,
               ohw_r, ohb_r, l1g_r, l1b_r, f1w_r, f1b_r, f2w_r, f2b_r,
               l2g_r, l2b_r,
               cawq_r, cawk_r, cawv_r, cawatt_r, cl1g_r, cl1b_r,
               cf1_r, cf2_r, cl2g_r, cl2b_r, hn_o):
    nf = nf_r[...]
    wv = wv_r[0] + wv_r[1]
    z16 = (zx_r[0] + zx_r[1])[:, :DH]
    z = jnp.dot(z16, e16_r[...], preferred_element_type=jnp.float32)
    h_attn = wv / (z + 1e-6)
    h = jnp.dot(h_attn, ohw_r[...], preferred_element_type=jnp.float32) + ohb_r[...]
    h = _ln(nf + h, l1g_r[...], l1b_r[...])
    h2 = jnp.maximum(
        jnp.dot(h, f1w_r[...], preferred_element_type=jnp.float32) + f1b_r[...], 0.0)
    h2 = jnp.dot(h2, f2w_r[...], preferred_element_type=jnp.float32) + f2b_r[...]
    h = _ln(h + h2, l2g_r[...], l2b_r[...])
    wfp = wfp_r[...]
    k = jnp.dot(wfp, cawk_r[...], preferred_element_type=jnp.float32)
    v = jnp.dot(wfp, cawv_r[...], preferred_element_type=jnp.float32)
    q = jnp.dot(h, cawq_r[...], preferred_element_type=jnp.float32)
    logits = jnp.dot(q, k.T, preferred_element_type=jnp.float32) * (
        1.0 / math.sqrt(D))
    logits = logits - jnp.max(logits, axis=-1, keepdims=True)
    logits = jnp.exp(logits)
    logits = logits / jnp.sum(logits, axis=-1, keepdims=True)
    att = jnp.dot(jnp.dot(logits, v, preferred_element_type=jnp.float32),
                  cawatt_r[...], preferred_element_type=jnp.float32)
    hn = _ln(h + att, cl1g_r[...], cl1b_r[...])
    ff = jnp.maximum(jnp.dot(hn, cf1_r[...], preferred_element_type=jnp.float32), 0.0)
    ff = jnp.dot(ff, cf2_r[...], preferred_element_type=jnp.float32)
    hn_o[...] = _ln(hn + ff, cl2g_r[...], cl2b_r[...])


def _node_post(nf, wv, zx, e16, wfp, p, bn=2000):
    grid = N // bn
    rspec = pl.BlockSpec((bn, D), lambda i: (i, 0))
    full = lambda *s: pl.BlockSpec(s, lambda i: (0,) * len(s))
    WDP = wfp.shape[1]
    pspec = pl.BlockSpec((NC, bn, D), lambda i: (0, i, 0))
    return pl.pallas_call(
        _node_body,
        grid=(grid,),
        in_specs=[rspec, pspec, pspec,
                  full(DH, D), full(NW, WDP),
                  full(D, D), full(D,), full(D,), full(D,),
                  full(D, 2 * D), full(2 * D,), full(2 * D, D), full(D,),
                  full(D,), full(D,),
                  full(D, D), full(WDP, D), full(WDP, D), full(D, D),
                  full(D,), full(D,),
                  full(D, 2 * D), full(2 * D, D), full(D,), full(D,)],
        out_specs=rspec,
        out_shape=jax.ShapeDtypeStruct((N, D), jnp.float32),
    )(nf, wv, zx, e16, wfp,
      p['OhW'], p['Ohb'], p['ln1h_g'], p['ln1h_b'],
      p['F1hW'], p['F1hb'], p['F2hW'], p['F2hb'], p['ln2h_g'], p['ln2h_b'],
      p['caWq'], p['_caWk_p'], p['_caWv_p'], p['caWatt'],
      p['ca_ln1_g'], p['ca_ln1_b'], p['caF1'], p['caF2'],
      p['ca_ln2_g'], p['ca_ln2_b'])


def kernel(node_feats, edge_feats, word_feats, edge_index, params):
    p = dict(params)
    src = edge_index[0]
    dst = edge_index[1]
    WDP = 384
    wfp = jnp.pad(word_feats, ((0, 0), (0, WDP - WD)))
    p['_caWk_p'] = jnp.pad(p['caWk'], ((0, WDP - WD), (0, 0)))
    p['_caWv_p'] = jnp.pad(p['caWv'], ((0, WDP - WD), (0, 0)))
    wq_scaled = p['WQ'] * (1.0 / math.sqrt(DH))
    c16 = (jnp.arange(D)[:, None] // DH == jnp.arange(DH)[None, :]).astype(
        jnp.float32)
    e16 = c16.T

    qh, kh, vh = _qkv(node_feats, wq_scaled, p['WK'], p['WV'])

    g = _gather_mul(kh, qh, src, dst)

    e_out, sc16 = _edge_dense(edge_feats, g, p, c16)

    wv_p = _scatter_seg(vh, src, dst, sc16)
    z_p = _scatter_z(dst, sc16)

    hn = _node_post(node_feats, wv_p, z_p, e16, wfp, p)
    return (hn, e_out)
